# baseline (speedup 1.0000x reference)
import functools

import jax
import jax.numpy as jnp
from jax import lax
from jax.experimental import pallas as pl
from jax.experimental.pallas import tpu as pltpu
from jax.experimental.pallas import tpu_sc as plsc

N = 10000
E = 320000
DIM = 128
LAYERS = 5

NC = 2
NS = 16
NW = NC * NS
EPW = E // NW
CHUNK = 80
NCHUNK = 128
EPP = NCHUNK * CHUNK
NROW = 4
NED = 8
RPS = 624
TAIL = N - NS * RPS


def _propagate_body(h_hbm, edata_hbm, zero_hbm, out_hbm, *refs):
    ebuf = refs[:NED]
    rows = refs[NED:NED + NROW]
    seme = refs[NED + NROW:2 * NED + NROW]
    semg = refs[2 * NED + NROW:2 * NED + 2 * NROW]
    sems = refs[2 * NED + 2 * NROW:2 * NED + 3 * NROW]
    semg2 = refs[2 * NED + 3 * NROW:2 * NED + 4 * NROW]
    acc = refs[-1]

    c = lax.axis_index("c")
    s = lax.axis_index("s")
    wid = c * NS + s

    pltpu.sync_copy(zero_hbm.at[pl.ds(s * RPS, RPS)], acc.at[pl.ds(s * RPS, RPS)])

    @pl.when(s == NS - 1)
    def _zero_tail():
        pltpu.sync_copy(zero_hbm.at[pl.ds(NS * RPS, TAIL)],
                        acc.at[pl.ds(NS * RPS, TAIL)])

    plsc.subcore_barrier()

    def fire_edata(g, m):
        pltpu.async_copy(edata_hbm.at[wid, g], ebuf[m], seme[m])

    def wait_edata(g, m):
        pltpu.make_async_copy(edata_hbm.at[wid, g], ebuf[m], seme[m]).wait()

    _H = CHUNK // 2

    def fire_gather(g, k, m):
        pltpu.async_copy(h_hbm.at[ebuf[m].at[0, pl.ds(0, _H)]],
                         rows[k].at[pl.ds(0, _H)], semg[k])
        pltpu.async_copy(h_hbm.at[ebuf[m].at[0, pl.ds(_H, _H)]],
                         rows[k].at[pl.ds(_H, _H)], semg2[k])

    def wait_gather(g, k, m):
        pltpu.make_async_copy(h_hbm.at[ebuf[m].at[0, pl.ds(0, _H)]],
                              rows[k].at[pl.ds(0, _H)], semg[k]).wait()
        pltpu.make_async_copy(h_hbm.at[ebuf[m].at[0, pl.ds(_H, _H)]],
                              rows[k].at[pl.ds(_H, _H)], semg2[k]).wait()

    def fire_scatter(g, k, m):
        pltpu.async_copy(rows[k], acc.at[ebuf[m].at[1]], sems[k], add=True)

    def wait_scatter(g, k, m):
        pltpu.make_async_copy(rows[k], acc.at[ebuf[m].at[1]], sems[k]).wait()

    two16 = jnp.full((16,), 2, jnp.int32)
    zero16 = jnp.full((16,), 0, jnp.int32)

    def scale(k, m):
        r = rows[k]
        eb = ebuf[m]

        def body(e, _):
            nb = plsc.bitcast(plsc.load_gather(eb, [two16, zero16 + e]),
                              jnp.float32)
            for j in range(DIM // 16):
                r[e, pl.ds(j * 16, 16)] = r[e, pl.ds(j * 16, 16)] * nb
            return 0

        lax.fori_loop(0, CHUNK, body, 0, unroll=2)

    for m in range(NED - 1):
        fire_edata(m, m)
    for j in range(NROW - 1):
        wait_edata(j, j)
        fire_gather(j, j, j)

    def octet(q, _):
        for i in range(NED):
            g = q * NED + i
            k = i % NROW
            m = i
            kn = (i + NROW - 1) % NROW
            mn = (i + NED - 1) % NED
            wait_gather(g, k, m)
            scale(k, m)

            @pl.when(g >= 1)
            def _ws():
                wait_scatter(g - 1, kn, mn)

            fire_scatter(g, k, m)

            @pl.when(g + NED - 1 < NCHUNK)
            def _fe():
                fire_edata(g + NED - 1, mn)

            @pl.when(g + NROW - 1 < NCHUNK)
            def _fg():
                wait_edata(g + NROW - 1, (i + NROW - 1) % NED)
                fire_gather(g + NROW - 1, kn, (i + NROW - 1) % NED)
        return 0

    lax.fori_loop(0, NCHUNK // NED, octet, 0)
    wait_scatter(NCHUNK - 1, (NCHUNK - 1) % NROW, (NCHUNK - 1) % NED)

    plsc.subcore_barrier()
    pltpu.sync_copy(acc.at[pl.ds(s * RPS, RPS)], out_hbm.at[c, pl.ds(s * RPS, RPS)])

    @pl.when(s == NS - 1)
    def _out_tail():
        pltpu.sync_copy(acc.at[pl.ds(NS * RPS, TAIL)],
                        out_hbm.at[c, pl.ds(NS * RPS, TAIL)])


def _propagate(h, edata, zeros):
    mesh = plsc.VectorSubcoreMesh(core_axis_name="c", subcore_axis_name="s")
    f = pl.kernel(
        _propagate_body,
        out_type=jax.ShapeDtypeStruct((NC, N, DIM), jnp.float32),
        mesh=mesh,
        compiler_params=pltpu.CompilerParams(needs_layout_passes=False),
        scratch_types=(
            [pltpu.VMEM((3, CHUNK), jnp.int32)] * NED
            + [pltpu.VMEM((CHUNK, DIM), jnp.float32)] * NROW
            + [pltpu.SemaphoreType.DMA] * (NED + 3 * NROW)
            + [pltpu.VMEM_SHARED((N, DIM), jnp.float32)]
        ),
    )
    return f(h, edata, zeros)


_RB = 2000


def _encode_body(x_ref, w_ref, o_ref):
    h = jnp.maximum(jnp.dot(x_ref[...], w_ref[...],
                            preferred_element_type=jnp.float32), 0.0)
    nn = jnp.sqrt(jnp.sum(h * h, axis=1, keepdims=True))
    o_ref[...] = h / jnp.maximum(nn, 1e-12)


def _encode(xp, wt):
    grid = (N // _RB,)
    return pl.pallas_call(
        _encode_body,
        grid=grid,
        in_specs=[
            pl.BlockSpec((_RB, 8), lambda i: (i, 0)),
            pl.BlockSpec((8, DIM), lambda i: (0, 0)),
        ],
        out_specs=pl.BlockSpec((_RB, DIM), lambda i: (i, 0)),
        out_shape=jax.ShapeDtypeStruct((N, DIM), jnp.float32),
    )(xp, wt)


def _gru_body(p_ref, h_ref, z_ref, wih_ref, whh_ref, bih_ref, bhh_ref,
              hn_ref, zn_ref):
    agg = p_ref[0] + p_ref[1]
    h = h_ref[...]
    gi = jnp.dot(agg, wih_ref[...], preferred_element_type=jnp.float32) + bih_ref[...]
    gh = jnp.dot(h, whh_ref[...], preferred_element_type=jnp.float32) + bhh_ref[...]
    r = jax.nn.sigmoid(gi[:, :DIM] + gh[:, :DIM])
    zg = jax.nn.sigmoid(gi[:, DIM:2 * DIM] + gh[:, DIM:2 * DIM])
    n = jnp.tanh(gi[:, 2 * DIM:] + r * gh[:, 2 * DIM:])
    hn = (1.0 - zg) * n + zg * h
    nn = jnp.sqrt(jnp.sum(hn * hn, axis=1, keepdims=True))
    hn = hn / jnp.maximum(nn, 1e-12)
    hn_ref[...] = hn
    zn_ref[...] = jnp.maximum(z_ref[...], hn)


def _gru(parts, h, z, wih_t, whh_t, bih, bhh):
    grid = (N // _RB,)
    return pl.pallas_call(
        _gru_body,
        grid=grid,
        in_specs=[
            pl.BlockSpec((NC, _RB, DIM), lambda i: (0, i, 0)),
            pl.BlockSpec((_RB, DIM), lambda i: (i, 0)),
            pl.BlockSpec((_RB, DIM), lambda i: (i, 0)),
            pl.BlockSpec((DIM, 3 * DIM), lambda i: (0, 0)),
            pl.BlockSpec((DIM, 3 * DIM), lambda i: (0, 0)),
            pl.BlockSpec((1, 3 * DIM), lambda i: (0, 0)),
            pl.BlockSpec((1, 3 * DIM), lambda i: (0, 0)),
        ],
        out_specs=[
            pl.BlockSpec((_RB, DIM), lambda i: (i, 0)),
            pl.BlockSpec((_RB, DIM), lambda i: (i, 0)),
        ],
        out_shape=[
            jax.ShapeDtypeStruct((N, DIM), jnp.float32),
            jax.ShapeDtypeStruct((N, DIM), jnp.float32),
        ],
    )(parts, h, z, wih_t, whh_t, bih, bhh)


def _decode_body(z_ref, w1_ref, w2_ref, o_ref):
    t = jnp.maximum(jnp.dot(z_ref[...], w1_ref[...],
                            preferred_element_type=jnp.float32), 0.0)
    o_ref[...] = jnp.sum(t * w2_ref[...], axis=1).reshape(1, 1, _RB)


def _decode(z, w1t, w2row):
    grid = (N // _RB,)
    out = pl.pallas_call(
        _decode_body,
        grid=grid,
        in_specs=[
            pl.BlockSpec((_RB, DIM), lambda i: (i, 0)),
            pl.BlockSpec((DIM, DIM // 2), lambda i: (0, 0)),
            pl.BlockSpec((1, DIM // 2), lambda i: (0, 0)),
        ],
        out_specs=pl.BlockSpec((1, 1, _RB), lambda i: (i, 0, 0)),
        out_shape=jax.ShapeDtypeStruct((N // _RB, 1, _RB), jnp.float32),
    )(z, w1t, w2row)
    return out.reshape(-1)


def kernel(x, edge_index, norm, W_enc, W_ih, W_hh, b_ih, b_hh, W_dec1, W_dec2):
    xp = jnp.pad(x, ((0, 0), (0, 5)))
    w_enc_t = jnp.pad(W_enc, ((0, 0), (0, 5))).T
    wih_t = W_ih.T
    whh_t = W_hh.T
    bih = b_ih.reshape(1, -1)
    bhh = b_hh.reshape(1, -1)
    w1t = W_dec1.T
    w2row = W_dec2.reshape(1, -1)
    pad = ((0, 0), (0, EPP - EPW))
    src2 = jnp.pad(edge_index[0].reshape(NW, EPW), pad)
    dst2 = jnp.pad(edge_index[1].reshape(NW, EPW), pad)
    nrm2 = jnp.pad(lax.bitcast_convert_type(norm, jnp.int32).reshape(NW, EPW),
                   pad)
    edata = jnp.stack([src2.reshape(NW, NCHUNK, CHUNK),
                       dst2.reshape(NW, NCHUNK, CHUNK),
                       nrm2.reshape(NW, NCHUNK, CHUNK)], axis=2)
    zeros = jnp.zeros((N, DIM), jnp.float32)

    h = _encode(xp, w_enc_t)
    z = h

    def layer(_, hz):
        h, z = hz
        parts = _propagate(h, edata, zeros)
        return _gru(parts, h, z, wih_t, whh_t, bih, bhh)

    h, z = lax.fori_loop(0, LAYERS - 1, layer, (h, z))
    y = _decode(z, w1t, w2row)
    return y


# speedup vs baseline: 1.6875x; 1.6875x over previous
import functools

import jax
import jax.numpy as jnp
from jax import lax
from jax.experimental import pallas as pl
from jax.experimental.pallas import tpu as pltpu
from jax.experimental.pallas import tpu_sc as plsc

N = 10000
E = 320000
DIM = 128
LAYERS = 5

NC = 2
NS = 16
NW = NC * NS
EPW = E // NW
CHUNK = 112
NCHUNK = 90
EPP = NCHUNK * CHUNK
NROW = 3
NED = 6
RPS = 624
TAIL = N - NS * RPS


def _propagate_body(h_hbm, edata_hbm, zero_hbm, out_hbm, *refs):
    ebuf = refs[:NED]
    rows = refs[NED:NED + NROW]
    seme = refs[NED + NROW:2 * NED + NROW]
    semg = refs[2 * NED + NROW:2 * NED + 2 * NROW]
    sems = refs[2 * NED + 2 * NROW:2 * NED + 3 * NROW]
    acc = refs[-1]

    c = lax.axis_index("c")
    s = lax.axis_index("s")
    wid = c * NS + s

    pltpu.sync_copy(zero_hbm.at[pl.ds(s * RPS, RPS)], acc.at[pl.ds(s * RPS, RPS)])

    @pl.when(s == NS - 1)
    def _zero_tail():
        pltpu.sync_copy(zero_hbm.at[pl.ds(NS * RPS, TAIL)],
                        acc.at[pl.ds(NS * RPS, TAIL)])

    plsc.subcore_barrier()

    def fire_edata(g, m):
        pltpu.async_copy(edata_hbm.at[wid, g], ebuf[m], seme[m])

    def wait_edata(g, m):
        pltpu.make_async_copy(edata_hbm.at[wid, g], ebuf[m], seme[m]).wait()

    def fire_gather(g, k, m):
        pltpu.async_copy(h_hbm.at[ebuf[m].at[0]], rows[k], semg[k])

    def wait_gather(g, k, m):
        pltpu.make_async_copy(h_hbm.at[ebuf[m].at[0]], rows[k], semg[k]).wait()

    def fire_scatter(g, k, m):
        pltpu.async_copy(rows[k], acc.at[ebuf[m].at[1]], sems[k], add=True)

    def wait_scatter(g, k, m):
        pltpu.make_async_copy(rows[k], acc.at[ebuf[m].at[1]], sems[k]).wait()

    two16 = jnp.full((16,), 2, jnp.int32)
    zero16 = jnp.full((16,), 0, jnp.int32)

    def scale(k, m):
        r = rows[k]
        eb = ebuf[m]

        def body(e, _):
            nb = plsc.bitcast(plsc.load_gather(eb, [two16, zero16 + e]),
                              jnp.float32)
            for j in range(DIM // 16):
                r[e, pl.ds(j * 16, 16)] = r[e, pl.ds(j * 16, 16)] * nb
            return 0

        lax.fori_loop(0, CHUNK, body, 0, unroll=2)

    for m in range(NED - 1):
        fire_edata(m, m)
    for j in range(NROW - 1):
        wait_edata(j, j)
        fire_gather(j, j, j)

    def octet(q, _):
        for i in range(NED):
            g = q * NED + i
            k = i % NROW
            m = i
            kn = (i + NROW - 1) % NROW
            mn = (i + NED - 1) % NED
            wait_gather(g, k, m)
            scale(k, m)

            @pl.when(g >= 1)
            def _ws():
                wait_scatter(g - 1, kn, mn)

            fire_scatter(g, k, m)

            @pl.when(g + NED - 1 < NCHUNK)
            def _fe():
                fire_edata(g + NED - 1, mn)

            @pl.when(g + NROW - 1 < NCHUNK)
            def _fg():
                wait_edata(g + NROW - 1, (i + NROW - 1) % NED)
                fire_gather(g + NROW - 1, kn, (i + NROW - 1) % NED)
        return 0

    lax.fori_loop(0, NCHUNK // NED, octet, 0)
    wait_scatter(NCHUNK - 1, (NCHUNK - 1) % NROW, (NCHUNK - 1) % NED)

    plsc.subcore_barrier()
    pltpu.sync_copy(acc.at[pl.ds(s * RPS, RPS)], out_hbm.at[c, pl.ds(s * RPS, RPS)])

    @pl.when(s == NS - 1)
    def _out_tail():
        pltpu.sync_copy(acc.at[pl.ds(NS * RPS, TAIL)],
                        out_hbm.at[c, pl.ds(NS * RPS, TAIL)])


def _propagate(h, edata, zeros):
    mesh = plsc.VectorSubcoreMesh(core_axis_name="c", subcore_axis_name="s")
    f = pl.kernel(
        _propagate_body,
        out_type=jax.ShapeDtypeStruct((NC, N, DIM), jnp.float32),
        mesh=mesh,
        compiler_params=pltpu.CompilerParams(needs_layout_passes=False),
        scratch_types=(
            [pltpu.VMEM((3, CHUNK), jnp.int32)] * NED
            + [pltpu.VMEM((CHUNK, DIM), jnp.float32)] * NROW
            + [pltpu.SemaphoreType.DMA] * (NED + 2 * NROW)
            + [pltpu.VMEM_SHARED((N, DIM), jnp.float32)]
        ),
    )
    return f(h, edata, zeros)


_RB = 2000


def _encode_body(x_ref, w_ref, o_ref):
    h = jnp.maximum(jnp.dot(x_ref[...], w_ref[...],
                            preferred_element_type=jnp.float32), 0.0)
    nn = jnp.sqrt(jnp.sum(h * h, axis=1, keepdims=True))
    o_ref[...] = h / jnp.maximum(nn, 1e-12)


def _encode(xp, wt):
    grid = (N // _RB,)
    return pl.pallas_call(
        _encode_body,
        grid=grid,
        in_specs=[
            pl.BlockSpec((_RB, 8), lambda i: (i, 0)),
            pl.BlockSpec((8, DIM), lambda i: (0, 0)),
        ],
        out_specs=pl.BlockSpec((_RB, DIM), lambda i: (i, 0)),
        out_shape=jax.ShapeDtypeStruct((N, DIM), jnp.float32),
    )(xp, wt)


def _gru_body(p_ref, h_ref, z_ref, wih_ref, whh_ref, bih_ref, bhh_ref,
              hn_ref, zn_ref):
    agg = p_ref[0] + p_ref[1]
    h = h_ref[...]
    gi = jnp.dot(agg, wih_ref[...], preferred_element_type=jnp.float32) + bih_ref[...]
    gh = jnp.dot(h, whh_ref[...], preferred_element_type=jnp.float32) + bhh_ref[...]
    r = jax.nn.sigmoid(gi[:, :DIM] + gh[:, :DIM])
    zg = jax.nn.sigmoid(gi[:, DIM:2 * DIM] + gh[:, DIM:2 * DIM])
    n = jnp.tanh(gi[:, 2 * DIM:] + r * gh[:, 2 * DIM:])
    hn = (1.0 - zg) * n + zg * h
    nn = jnp.sqrt(jnp.sum(hn * hn, axis=1, keepdims=True))
    hn = hn / jnp.maximum(nn, 1e-12)
    hn_ref[...] = hn
    zn_ref[...] = jnp.maximum(z_ref[...], hn)


def _gru(parts, h, z, wih_t, whh_t, bih, bhh):
    grid = (N // _RB,)
    return pl.pallas_call(
        _gru_body,
        grid=grid,
        in_specs=[
            pl.BlockSpec((NC, _RB, DIM), lambda i: (0, i, 0)),
            pl.BlockSpec((_RB, DIM), lambda i: (i, 0)),
            pl.BlockSpec((_RB, DIM), lambda i: (i, 0)),
            pl.BlockSpec((DIM, 3 * DIM), lambda i: (0, 0)),
            pl.BlockSpec((DIM, 3 * DIM), lambda i: (0, 0)),
            pl.BlockSpec((1, 3 * DIM), lambda i: (0, 0)),
            pl.BlockSpec((1, 3 * DIM), lambda i: (0, 0)),
        ],
        out_specs=[
            pl.BlockSpec((_RB, DIM), lambda i: (i, 0)),
            pl.BlockSpec((_RB, DIM), lambda i: (i, 0)),
        ],
        out_shape=[
            jax.ShapeDtypeStruct((N, DIM), jnp.float32),
            jax.ShapeDtypeStruct((N, DIM), jnp.float32),
        ],
    )(parts, h, z, wih_t, whh_t, bih, bhh)


def _decode_body(z_ref, w1_ref, w2_ref, o_ref):
    t = jnp.maximum(jnp.dot(z_ref[...], w1_ref[...],
                            preferred_element_type=jnp.float32), 0.0)
    o_ref[...] = jnp.sum(t * w2_ref[...], axis=1).reshape(1, 1, _RB)


def _decode(z, w1t, w2row):
    grid = (N // _RB,)
    out = pl.pallas_call(
        _decode_body,
        grid=grid,
        in_specs=[
            pl.BlockSpec((_RB, DIM), lambda i: (i, 0)),
            pl.BlockSpec((DIM, DIM // 2), lambda i: (0, 0)),
            pl.BlockSpec((1, DIM // 2), lambda i: (0, 0)),
        ],
        out_specs=pl.BlockSpec((1, 1, _RB), lambda i: (i, 0, 0)),
        out_shape=jax.ShapeDtypeStruct((N // _RB, 1, _RB), jnp.float32),
    )(z, w1t, w2row)
    return out.reshape(-1)


def kernel(x, edge_index, norm, W_enc, W_ih, W_hh, b_ih, b_hh, W_dec1, W_dec2):
    xp = jnp.pad(x, ((0, 0), (0, 5)))
    w_enc_t = jnp.pad(W_enc, ((0, 0), (0, 5))).T
    wih_t = W_ih.T
    whh_t = W_hh.T
    bih = b_ih.reshape(1, -1)
    bhh = b_hh.reshape(1, -1)
    w1t = W_dec1.T
    w2row = W_dec2.reshape(1, -1)
    pad = ((0, 0), (0, EPP - EPW))
    src2 = jnp.pad(edge_index[0].reshape(NW, EPW), pad)
    dst2 = jnp.pad(edge_index[1].reshape(NW, EPW), pad)
    nrm2 = jnp.pad(lax.bitcast_convert_type(norm, jnp.int32).reshape(NW, EPW),
                   pad)
    edata = jnp.stack([src2.reshape(NW, NCHUNK, CHUNK),
                       dst2.reshape(NW, NCHUNK, CHUNK),
                       nrm2.reshape(NW, NCHUNK, CHUNK)], axis=2)
    zeros = jnp.zeros((N, DIM), jnp.float32)

    h = _encode(xp, w_enc_t)
    z = h

    def layer(_, hz):
        h, z = hz
        parts = _propagate(h, edata, zeros)
        return _gru(parts, h, z, wih_t, whh_t, bih, bhh)

    h, z = lax.fori_loop(0, LAYERS - 1, layer, (h, z))
    y = _decode(z, w1t, w2row)
    return y
